# baseline (speedup 1.0000x reference)
import jax
import jax.numpy as jnp
from jax.experimental import pallas as pl
from jax.experimental.pallas import tpu as pltpu

_BLOCK = 256


def _body(state_ref, mask_ref, wt_ref, b_ref, emb_ref, g_ref,
          logits_ref, policy_ref, idx_ref, out_ref):
    state = state_ref[...]
    logits = jnp.dot(state, wt_ref[...],
                     preferred_element_type=jnp.float32) + b_ref[...]
    m = mask_ref[...]
    l_min = jnp.min(logits, axis=-1, keepdims=True)
    lg = jnp.where(m, logits, l_min)
    lg = lg - jnp.max(lg, axis=-1, keepdims=True)
    lg = lg * m.astype(lg.dtype)
    el = jnp.where(m, jnp.exp(lg), 0.0)
    policy = el / jnp.sum(el, axis=-1, keepdims=True)
    score = g_ref[...] + jnp.log(policy)
    s0, s1, s2 = score[:, 0:1], score[:, 1:2], score[:, 2:3]
    idx = jnp.where((s0 >= s1) & (s0 >= s2), 0,
                    jnp.where(s1 >= s2, 1, 2)).astype(jnp.int32)
    e0, e1, e2 = emb_ref[0:1, :], emb_ref[1:2, :], emb_ref[2:3, :]
    esel = jnp.where(idx == 0, e0, jnp.where(idx == 1, e1, e2))
    logits_ref[...] = logits
    policy_ref[...] = policy
    idx_ref[...] = idx
    out_ref[...] = esel * (0.5 * jnp.tanh(state * 0.5) + 0.5)


def kernel(state, action_type_mask, W, b, emb):
    B, T, D = state.shape
    A = W.shape[0]
    BT = B * T
    state2 = state.reshape(BT, D)
    mask2 = action_type_mask.reshape(BT, A)
    g = jax.random.gumbel(jax.random.key(42), (BT, A), jnp.float32)

    n_blocks = BT // _BLOCK
    logits2, policy2, idx2, out2 = pl.pallas_call(
        _body,
        grid=(n_blocks,),
        in_specs=[
            pl.BlockSpec((_BLOCK, D), lambda i: (i, 0)),
            pl.BlockSpec((_BLOCK, A), lambda i: (i, 0)),
            pl.BlockSpec((D, A), lambda i: (0, 0)),
            pl.BlockSpec((1, A), lambda i: (0, 0)),
            pl.BlockSpec((A, D), lambda i: (0, 0)),
            pl.BlockSpec((_BLOCK, A), lambda i: (i, 0)),
        ],
        out_specs=[
            pl.BlockSpec((_BLOCK, A), lambda i: (i, 0)),
            pl.BlockSpec((_BLOCK, A), lambda i: (i, 0)),
            pl.BlockSpec((_BLOCK, 1), lambda i: (i, 0)),
            pl.BlockSpec((_BLOCK, D), lambda i: (i, 0)),
        ],
        out_shape=[
            jax.ShapeDtypeStruct((BT, A), jnp.float32),
            jax.ShapeDtypeStruct((BT, A), jnp.float32),
            jax.ShapeDtypeStruct((BT, 1), jnp.int32),
            jax.ShapeDtypeStruct((BT, D), jnp.float32),
        ],
    )(state2, mask2, W.T, b.reshape(1, A), emb, g)

    return (logits2.reshape(B, T, A),
            policy2.reshape(B, T, A),
            idx2.reshape(B, T),
            out2.reshape(B, T, D))


# speedup vs baseline: 1.0813x; 1.0813x over previous
import jax
import jax.numpy as jnp
from jax.experimental import pallas as pl
from jax.experimental.pallas import tpu as pltpu

_BLOCK = 1024
_CHUNK = 256


def _body(state_ref, mask_ref, wt_ref, b_ref, emb_ref, g_ref,
          logits_ref, policy_ref, idx_ref, out_ref):
    logits = jnp.dot(state_ref[...], wt_ref[...],
                     preferred_element_type=jnp.float32) + b_ref[...]
    m = mask_ref[...]
    l_min = jnp.min(logits, axis=-1, keepdims=True)
    lg = jnp.where(m, logits, l_min)
    lg = lg - jnp.max(lg, axis=-1, keepdims=True)
    lg = lg * m.astype(lg.dtype)
    el = jnp.where(m, jnp.exp(lg), 0.0)
    policy = el / jnp.sum(el, axis=-1, keepdims=True)
    score = g_ref[...] + jnp.log(policy)
    s0, s1, s2 = score[:, 0:1], score[:, 1:2], score[:, 2:3]
    idx = jnp.where((s0 >= s1) & (s0 >= s2), 0,
                    jnp.where(s1 >= s2, 1, 2)).astype(jnp.int32)
    logits_ref[...] = logits
    policy_ref[...] = policy
    idx_ref[...] = idx
    lanes = jax.lax.broadcasted_iota(jnp.int32, (1, 3), 1)
    oh = (idx == lanes).astype(jnp.float32)
    d = state_ref.shape[1]
    for c in range(0, d, _CHUNK):
        sl = pl.ds(c, _CHUNK)
        esel = jnp.dot(oh, emb_ref[:, sl],
                       preferred_element_type=jnp.float32)
        out_ref[:, sl] = esel * (
            0.5 * jnp.tanh(state_ref[:, sl] * 0.5) + 0.5)


def kernel(state, action_type_mask, W, b, emb):
    B, T, D = state.shape
    A = W.shape[0]
    BT = B * T
    state2 = state.reshape(BT, D)
    mask2 = action_type_mask.reshape(BT, A)
    g = jax.random.gumbel(jax.random.key(42), (BT, A), jnp.float32)

    n_blocks = BT // _BLOCK
    logits2, policy2, idx2, out2 = pl.pallas_call(
        _body,
        grid=(n_blocks,),
        in_specs=[
            pl.BlockSpec((_BLOCK, D), lambda i: (i, 0)),
            pl.BlockSpec((_BLOCK, A), lambda i: (i, 0)),
            pl.BlockSpec((D, A), lambda i: (0, 0)),
            pl.BlockSpec((1, A), lambda i: (0, 0)),
            pl.BlockSpec((A, D), lambda i: (0, 0)),
            pl.BlockSpec((_BLOCK, A), lambda i: (i, 0)),
        ],
        out_specs=[
            pl.BlockSpec((_BLOCK, A), lambda i: (i, 0)),
            pl.BlockSpec((_BLOCK, A), lambda i: (i, 0)),
            pl.BlockSpec((_BLOCK, 1), lambda i: (i, 0)),
            pl.BlockSpec((_BLOCK, D), lambda i: (i, 0)),
        ],
        out_shape=[
            jax.ShapeDtypeStruct((BT, A), jnp.float32),
            jax.ShapeDtypeStruct((BT, A), jnp.float32),
            jax.ShapeDtypeStruct((BT, 1), jnp.int32),
            jax.ShapeDtypeStruct((BT, D), jnp.float32),
        ],
    )(state2, mask2, W.T, b.reshape(1, A), emb, g)

    return (logits2.reshape(B, T, A),
            policy2.reshape(B, T, A),
            idx2.reshape(B, T),
            out2.reshape(B, T, D))


# speedup vs baseline: 1.0996x; 1.0170x over previous
import jax
import jax.numpy as jnp
from jax.experimental import pallas as pl
from jax.experimental.pallas import tpu as pltpu

_BLOCK = 1024
_CHUNK = 64


def _body(state_ref, mask_ref, wt_ref, b_ref, emb_ref, g_ref,
          logits_ref, policy_ref, idx_ref, out_ref):
    logits = jnp.dot(state_ref[...], wt_ref[...],
                     preferred_element_type=jnp.float32) + b_ref[...]
    m = mask_ref[...]
    l_min = jnp.min(logits, axis=-1, keepdims=True)
    lg = jnp.where(m, logits, l_min)
    lg = lg - jnp.max(lg, axis=-1, keepdims=True)
    lg = lg * m.astype(lg.dtype)
    el = jnp.where(m, jnp.exp(lg), 0.0)
    policy = el / jnp.sum(el, axis=-1, keepdims=True)
    score = g_ref[...] + jnp.log(policy)
    s0, s1, s2 = score[:, 0:1], score[:, 1:2], score[:, 2:3]
    idx = jnp.where((s0 >= s1) & (s0 >= s2), 0,
                    jnp.where(s1 >= s2, 1, 2)).astype(jnp.int32)
    logits_ref[...] = logits
    policy_ref[...] = policy
    idx_ref[...] = idx
    e0, e1, e2 = emb_ref[0:1, :], emb_ref[1:2, :], emb_ref[2:3, :]

    def _tail(r, carry):
        rs = pl.ds(r * _CHUNK, _CHUNK)
        i0 = idx_ref[rs, :] == 0
        i1 = idx_ref[rs, :] == 1
        esel = jnp.where(i0, e0, jnp.where(i1, e1, e2))
        u = esel * 0.5
        out_ref[rs, :] = u * jnp.tanh(state_ref[rs, :] * 0.5) + u
        return carry

    jax.lax.fori_loop(0, _BLOCK // _CHUNK, _tail, 0)


def kernel(state, action_type_mask, W, b, emb):
    B, T, D = state.shape
    A = W.shape[0]
    BT = B * T
    state2 = state.reshape(BT, D)
    mask2 = action_type_mask.reshape(BT, A)
    g = jax.random.gumbel(jax.random.key(42), (BT, A), jnp.float32)

    n_blocks = BT // _BLOCK
    logits2, policy2, idx2, out2 = pl.pallas_call(
        _body,
        grid=(n_blocks,),
        in_specs=[
            pl.BlockSpec((_BLOCK, D), lambda i: (i, 0)),
            pl.BlockSpec((_BLOCK, A), lambda i: (i, 0)),
            pl.BlockSpec((D, A), lambda i: (0, 0)),
            pl.BlockSpec((1, A), lambda i: (0, 0)),
            pl.BlockSpec((A, D), lambda i: (0, 0)),
            pl.BlockSpec((_BLOCK, A), lambda i: (i, 0)),
        ],
        out_specs=[
            pl.BlockSpec((_BLOCK, A), lambda i: (i, 0)),
            pl.BlockSpec((_BLOCK, A), lambda i: (i, 0)),
            pl.BlockSpec((_BLOCK, 1), lambda i: (i, 0)),
            pl.BlockSpec((_BLOCK, D), lambda i: (i, 0)),
        ],
        out_shape=[
            jax.ShapeDtypeStruct((BT, A), jnp.float32),
            jax.ShapeDtypeStruct((BT, A), jnp.float32),
            jax.ShapeDtypeStruct((BT, 1), jnp.int32),
            jax.ShapeDtypeStruct((BT, D), jnp.float32),
        ],
    )(state2, mask2, W.T, b.reshape(1, A), emb, g)

    return (logits2.reshape(B, T, A),
            policy2.reshape(B, T, A),
            idx2.reshape(B, T),
            out2.reshape(B, T, D))


# speedup vs baseline: 1.1678x; 1.0620x over previous
import jax
import jax.numpy as jnp
from jax.experimental import pallas as pl
from jax.experimental.pallas import tpu as pltpu

_BLOCK = 1024


def _body(state_ref, mask_ref, wt_ref, b_ref, emb_ref, g_ref,
          logits_ref, policy_ref, idx_ref, out_ref):
    state = state_ref[...]
    logits = jnp.dot(state, wt_ref[...],
                     preferred_element_type=jnp.float32) + b_ref[...]
    m = mask_ref[...]
    l_min = jnp.min(logits, axis=-1, keepdims=True)
    lg = jnp.where(m, logits, l_min)
    lg = lg - jnp.max(lg, axis=-1, keepdims=True)
    lg = lg * m.astype(lg.dtype)
    el = jnp.where(m, jnp.exp(lg), 0.0)
    policy = el / jnp.sum(el, axis=-1, keepdims=True)
    score = g_ref[...] + jnp.log(policy)
    s0, s1, s2 = score[:, 0:1], score[:, 1:2], score[:, 2:3]
    idx = jnp.where((s0 >= s1) & (s0 >= s2), 0,
                    jnp.where(s1 >= s2, 1, 2)).astype(jnp.int32)
    e0, e1, e2 = emb_ref[0:1, :], emb_ref[1:2, :], emb_ref[2:3, :]
    esel = jnp.where(idx == 0, e0, jnp.where(idx == 1, e1, e2))
    logits_ref[...] = logits
    policy_ref[...] = policy
    idx_ref[...] = idx
    out_ref[...] = esel * (0.5 * jnp.tanh(state * 0.5) + 0.5)


def kernel(state, action_type_mask, W, b, emb):
    B, T, D = state.shape
    A = W.shape[0]
    BT = B * T
    state2 = state.reshape(BT, D)
    mask2 = action_type_mask.reshape(BT, A)
    g = jax.random.gumbel(jax.random.key(42), (BT, A), jnp.float32)

    n_blocks = BT // _BLOCK
    logits2, policy2, idx2, out2 = pl.pallas_call(
        _body,
        grid=(n_blocks,),
        in_specs=[
            pl.BlockSpec((_BLOCK, D), lambda i: (i, 0)),
            pl.BlockSpec((_BLOCK, A), lambda i: (i, 0)),
            pl.BlockSpec((D, A), lambda i: (0, 0)),
            pl.BlockSpec((1, A), lambda i: (0, 0)),
            pl.BlockSpec((A, D), lambda i: (0, 0)),
            pl.BlockSpec((_BLOCK, A), lambda i: (i, 0)),
        ],
        out_specs=[
            pl.BlockSpec((_BLOCK, A), lambda i: (i, 0)),
            pl.BlockSpec((_BLOCK, A), lambda i: (i, 0)),
            pl.BlockSpec((_BLOCK, 1), lambda i: (i, 0)),
            pl.BlockSpec((_BLOCK, D), lambda i: (i, 0)),
        ],
        out_shape=[
            jax.ShapeDtypeStruct((BT, A), jnp.float32),
            jax.ShapeDtypeStruct((BT, A), jnp.float32),
            jax.ShapeDtypeStruct((BT, 1), jnp.int32),
            jax.ShapeDtypeStruct((BT, D), jnp.float32),
        ],
    )(state2, mask2, W.T, b.reshape(1, A), emb, g)

    return (logits2.reshape(B, T, A),
            policy2.reshape(B, T, A),
            idx2.reshape(B, T),
            out2.reshape(B, T, D))
